# baseline (speedup 1.0000x reference)
import functools

import jax
import jax.numpy as jnp
from jax import lax
from jax.experimental import pallas as pl
from jax.experimental.pallas import tpu as pltpu
from jax.experimental.pallas import tpu_sc as plsc

N = 10000
D = 128
F32 = jnp.float32
CHUNK = 128


def _sc_mesh():
    return plsc.VectorSubcoreMesh(core_axis_name="c", subcore_axis_name="s")


def _on_slot(pt, fn):
    @pl.when(pt == 0)
    def _():
        fn(0)

    @pl.when(pt == 1)
    def _():
        fn(1)


_SC_PARAMS = pltpu.CompilerParams(use_tc_tiling_on_sc=False)


@functools.lru_cache(maxsize=None)
def _make_segsum(bs, d, dt=F32):
    dh = d // 2
    npr = N // 16
    tails = [b % CHUNK for b in bs]
    scratch = [
        pltpu.VMEM_SHARED((N, dh), dt),
        pltpu.VMEM((CHUNK, dh), dt),
        pltpu.VMEM((CHUNK, dh), dt),
        pltpu.VMEM((CHUNK,), jnp.int32),
        pltpu.VMEM((CHUNK,), jnp.int32),
        pltpu.SemaphoreType.DMA,
        pltpu.SemaphoreType.DMA,
        pltpu.SemaphoreType.DMA,
    ]
    for t in tails:
        if t:
            scratch.append(pltpu.VMEM((t, dh), dt))
            scratch.append(pltpu.VMEM((t,), jnp.int32))

    @functools.partial(
        pl.kernel,
        out_type=jax.ShapeDtypeStruct((N, d), dt),
        mesh=_sc_mesh(),
        scratch_types=scratch,
        compiler_params=_SC_PARAMS,
    )
    def seg(*refs):
        zero_hbm = refs[0]
        data = refs[1:1 + 2 * len(bs)]
        out = refs[1 + 2 * len(bs)]
        rest = list(refs[2 + 2 * len(bs):])
        acc = rest[0]
        rbuf = rest[1:3]
        ibuf = rest[3:5]
        lsem = rest[5]
        ssem = rest[6:8]
        tbufs = rest[8:]
        c = lax.axis_index("c")
        s = lax.axis_index("s")
        pltpu.sync_copy(zero_hbm.at[pl.ds(s * npr, npr), pl.ds(0, dh)],
                        acc.at[pl.ds(s * npr, npr)])
        plsc.subcore_barrier()
        ti = 0
        for j, b in enumerate(bs):
            rows_hbm = data[2 * j]
            idx_hbm = data[2 * j + 1]
            nch = b // CHUNK
            tail = tails[j]

            def start_loads(i, p, rows_hbm=rows_hbm, idx_hbm=idx_hbm):
                base = i * CHUNK
                pltpu.async_copy(
                    rows_hbm.at[pl.ds(base, CHUNK), pl.ds(c * dh, dh)],
                    rbuf[p], lsem)
                pltpu.async_copy(idx_hbm.at[pl.ds(base, CHUNK)], ibuf[p],
                                 lsem)

            def wait_loads(p):
                pltpu.make_async_copy(
                    rows_hbm.at[pl.ds(0, CHUNK), pl.ds(0, dh)],
                    rbuf[p], lsem).wait()
                pltpu.make_async_copy(idx_hbm.at[pl.ds(0, CHUNK)], ibuf[p],
                                      lsem).wait()

            def wait_scat(p):
                pltpu.make_async_copy(rbuf[p], acc.at[ibuf[p]],
                                      ssem[p]).wait()

            nj = jnp.maximum(0, (nch - s + 15) // 16)

            @pl.when(nj > 0)
            def _():
                start_loads(s, 0)

            @pl.loop(0, nj)
            def _(jj):
                p = lax.rem(jj, 2)
                _on_slot(p, wait_loads)
                _on_slot(p, lambda q: pltpu.async_copy(
                    rbuf[q], acc.at[ibuf[q]], ssem[q], add=True))

                @pl.when(jj + 1 < nj)
                def _():
                    @pl.when(jj >= 1)
                    def _():
                        _on_slot(1 - p, wait_scat)

                    _on_slot(1 - p,
                             lambda q: start_loads(s + (jj + 1) * 16, q))

            @pl.when(nj >= 2)
            def _():
                _on_slot(lax.rem(nj, 2), wait_scat)

            @pl.when(nj >= 1)
            def _():
                _on_slot(lax.rem(nj + 1, 2), wait_scat)

            if tail:
                trbuf, tibuf = tbufs[ti], tbufs[ti + 1]
                ti += 2

                @pl.when(s == nch % 16)
                def _():
                    base = nch * CHUNK
                    pltpu.sync_copy(
                        rows_hbm.at[pl.ds(base, tail), pl.ds(c * dh, dh)],
                        trbuf)
                    pltpu.sync_copy(idx_hbm.at[pl.ds(base, tail)], tibuf)
                    pltpu.sync_copy(trbuf, acc.at[tibuf], add=True)
        plsc.subcore_barrier()
        pltpu.sync_copy(acc.at[pl.ds(s * npr, npr)],
                        out.at[pl.ds(s * npr, npr), pl.ds(c * dh, dh)])

    return seg


@functools.lru_cache(maxsize=None)
def _make_gather(b, d, dt=F32):
    nch = b // CHUNK
    tail = b % CHUNK
    scratch = [
        pltpu.VMEM((CHUNK, d), dt),
        pltpu.VMEM((CHUNK, d), dt),
        pltpu.VMEM((CHUNK,), jnp.int32),
        pltpu.VMEM((CHUNK,), jnp.int32),
        pltpu.SemaphoreType.DMA,
        pltpu.SemaphoreType.DMA,
    ]
    if tail:
        scratch.append(pltpu.VMEM((tail, d), dt))
        scratch.append(pltpu.VMEM((tail,), jnp.int32))

    @functools.partial(
        pl.kernel,
        out_type=jax.ShapeDtypeStruct((b, d), dt),
        mesh=_sc_mesh(),
        scratch_types=scratch,
        compiler_params=_SC_PARAMS,
    )
    def gat(*refs):
        table_hbm, idx_hbm, out_hbm = refs[:3]
        rest = list(refs[3:])
        rbuf = rest[0:2]
        ibuf = rest[2:4]
        gsem = rest[4:6]
        c = lax.axis_index("c")
        s = lax.axis_index("s")
        w = s * 2 + c
        nj = jnp.maximum(0, (nch - w + 31) // 32)

        def load_idx(t, q):
            pltpu.sync_copy(idx_hbm.at[pl.ds((w + 32 * t) * CHUNK, CHUNK)],
                            ibuf[q])

        def start_gather(q):
            pltpu.async_copy(table_hbm.at[ibuf[q]], rbuf[q], gsem[q])

        def wait_gather(q):
            pltpu.make_async_copy(table_hbm.at[ibuf[q]], rbuf[q],
                                  gsem[q]).wait()

        @pl.when(nj > 0)
        def _():
            load_idx(0, 0)
            start_gather(0)

        @pl.loop(0, nj)
        def _(jj):
            p = lax.rem(jj, 2)

            @pl.when(jj + 1 < nj)
            def _():
                _on_slot(1 - p, lambda q: load_idx(jj + 1, q))

            _on_slot(p, wait_gather)

            @pl.when(jj + 1 < nj)
            def _():
                _on_slot(1 - p, start_gather)

            _on_slot(p, lambda q: pltpu.sync_copy(
                rbuf[q],
                out_hbm.at[pl.ds((w + 32 * jj) * CHUNK, CHUNK)]))

        if tail:
            trbuf, tibuf = rest[6], rest[7]

            @pl.when(w == nch % 32)
            def _():
                base = nch * CHUNK
                pltpu.sync_copy(idx_hbm.at[pl.ds(base, tail)], tibuf)
                pltpu.async_copy(table_hbm.at[tibuf], trbuf, gsem[0]).wait()
                pltpu.sync_copy(trbuf, out_hbm.at[pl.ds(base, tail)])

    return gat


GCH = 120


@functools.lru_cache(maxsize=None)
def _make_gather_x(b, k):
    nch = b // GCH
    tail = b % GCH
    assert tail % k == 0
    scratch = [
        pltpu.VMEM((GCH, D), F32),
        pltpu.VMEM((GCH, D), F32),
        pltpu.VMEM((GCH, D), F32),
        pltpu.VMEM((GCH, D), F32),
        pltpu.VMEM((GCH,), jnp.int32),
        pltpu.VMEM((GCH,), jnp.int32),
        pltpu.SemaphoreType.DMA,
        pltpu.SemaphoreType.DMA,
    ]

    @functools.partial(
        pl.kernel,
        out_type=(jax.ShapeDtypeStruct((b, D), F32),
                  jax.ShapeDtypeStruct((b, D), F32)),
        mesh=_sc_mesh(),
        scratch_types=scratch,
        compiler_params=_SC_PARAMS,
    )
    def gatx(*refs):
        table_hbm, idx_hbm, oloc_hbm, obc_hbm = refs[:4]
        rest = list(refs[4:])
        rbuf = rest[0:2]
        bbuf = rest[2:4]
        ibuf = rest[4:6]
        gsem = rest[6:8]
        c = lax.axis_index("c")
        s = lax.axis_index("s")
        w = s * 2 + c
        nj = jnp.maximum(0, (nch - w + 31) // 32)

        def load_idx(t, q):
            pltpu.sync_copy(idx_hbm.at[pl.ds((w + 32 * t) * GCH, GCH)],
                            ibuf[q])

        def start_gather(q):
            pltpu.async_copy(table_hbm.at[ibuf[q]], rbuf[q], gsem[q])

        def wait_gather(q):
            pltpu.make_async_copy(table_hbm.at[ibuf[q]], rbuf[q],
                                  gsem[q]).wait()

        def bcast_compute(q, nrows):
            for g in range(nrows // k):
                for c8 in range(D // 16):
                    sl = pl.ds(c8 * 16, 16)
                    acc = rbuf[q][g * k, sl]
                    for r in range(1, k):
                        acc = acc + rbuf[q][g * k + r, sl]
                    for r in range(k):
                        bbuf[q][g * k + r, sl] = acc

        def write_out(q, t):
            base = (w + 32 * t) * GCH
            pltpu.sync_copy(rbuf[q], oloc_hbm.at[pl.ds(base, GCH)])
            pltpu.sync_copy(bbuf[q], obc_hbm.at[pl.ds(base, GCH)])

        @pl.when(nj > 0)
        def _():
            load_idx(0, 0)
            start_gather(0)

        @pl.loop(0, nj)
        def _(jj):
            p = lax.rem(jj, 2)

            @pl.when(jj + 1 < nj)
            def _():
                _on_slot(1 - p, lambda q: load_idx(jj + 1, q))

            _on_slot(p, wait_gather)

            @pl.when(jj + 1 < nj)
            def _():
                _on_slot(1 - p, start_gather)

            _on_slot(p, lambda q: bcast_compute(q, GCH))
            _on_slot(p, lambda q: write_out(q, jj))

        if tail:

            @pl.when(w == nch % 32)
            def _():
                base = nch * GCH
                pltpu.sync_copy(idx_hbm.at[pl.ds(base, tail)],
                                ibuf[0].at[pl.ds(0, tail)])
                pltpu.async_copy(
                    table_hbm.at[ibuf[0].at[pl.ds(0, tail)]],
                    rbuf[0].at[pl.ds(0, tail)], gsem[0]).wait()
                bcast_compute(0, tail)
                pltpu.sync_copy(rbuf[0].at[pl.ds(0, tail)],
                                oloc_hbm.at[pl.ds(base, tail)])
                pltpu.sync_copy(bbuf[0].at[pl.ds(0, tail)],
                                obc_hbm.at[pl.ds(base, tail)])

    return gatx


def _group_bcast(x, k):
    r, ch = x.shape
    g = jnp.sum(x.reshape(r // k, k, ch), axis=1, keepdims=True)
    return jnp.broadcast_to(g, (r // k, k, ch)).reshape(r, ch)


def _group_sum_roll(y, k):
    r, ch = y.shape
    pos = lax.rem(lax.broadcasted_iota(jnp.int32, (r, ch), 0), k)
    acc = y
    for d in range(1, k):
        up = pltpu.roll(y, r - d, 0)
        down = pltpu.roll(y, d, 0)
        acc = acc + jnp.where(pos < k - d, up, 0.0)
        acc = acc + jnp.where(pos >= d, down, 0.0)
    return acc


def _full(shape):
    nd = len(shape)
    return pl.BlockSpec(shape, lambda i, _n=nd: (0,) * _n)


def _rows(rb, ch):
    return pl.BlockSpec((rb, ch), lambda i: (i, 0))


def _dot(a, b):
    return jnp.dot(a.astype(jnp.bfloat16), b.astype(jnp.bfloat16),
                   preferred_element_type=F32)


@functools.lru_cache(maxsize=None)
def _make_xbuild(b, k, rb, odt=F32):
    nb = b // rb

    def body(x_ref, o_ref):
        x = x_ref[...]
        o_ref[...] = jnp.concatenate(
            [x, _group_sum_roll(x, k)], axis=1).astype(odt)

    return pl.pallas_call(
        body,
        grid=(nb,),
        in_specs=[_rows(rb, D)],
        out_specs=_rows(rb, 2 * D),
        out_shape=jax.ShapeDtypeStruct((b, 2 * D), odt),
    )


@functools.lru_cache(maxsize=None)
def _make_cycle_head(b, k, rb, idt=F32, odt=F32):
    nb = b // rb

    def body(xa_ref, xb_ref, rep_ref, wl_ref, bl_ref, w1_ref, sc_ref,
             e2c_ref, h1_ref, mom_ref):
        i = pl.program_id(0)
        x32 = jnp.concatenate([xa_ref[...], xb_ref[...]],
                              axis=1).astype(F32)
        y = _dot(x32, wl_ref[2 * D:, :])
        e2c = (_dot(x32, wl_ref[: 2 * D, :]) + _group_sum_roll(y, k)
               + bl_ref[0:1, :])
        e2c_ref[...] = e2c.astype(odt)
        pre = e2c + sc_ref[0, 0] * rep_ref[...]
        h1 = _dot(pre, w1_ref[...])
        h1_ref[...] = h1.astype(odt)

        @pl.when(i == 0)
        def _():
            mom_ref[...] = jnp.zeros_like(mom_ref)

        mom_ref[0:1, :] += jnp.sum(h1, axis=0, keepdims=True)
        mom_ref[1:2, :] += jnp.sum(h1 * h1, axis=0, keepdims=True)

    return pl.pallas_call(
        body,
        grid=(nb,),
        in_specs=[
            _rows(rb, D),
            _rows(rb, D),
            _rows(rb, D),
            _full((4 * D, D)),
            _full((1, D)),
            _full((D, 2 * D)),
            pl.BlockSpec(memory_space=pltpu.SMEM),
        ],
        out_specs=[
            _rows(rb, D),
            _rows(rb, 2 * D),
            _full((8, 2 * D)),
        ],
        out_shape=[
            jax.ShapeDtypeStruct((b, D), odt),
            jax.ShapeDtypeStruct((b, 2 * D), odt),
            jax.ShapeDtypeStruct((8, 2 * D), F32),
        ],
    )


@functools.lru_cache(maxsize=None)
def _make_edge_head(b, rb, idt=F32, odt=F32):
    nb = b // rb

    def body(la_ref, lb_ref, rep_ref, wel_ref, bel_ref, w1_ref, sc_ref,
             h1_ref, mom_ref):
        i = pl.program_id(0)
        x = jnp.concatenate([la_ref[...], lb_ref[...]], axis=1).astype(F32)
        y = _dot(x, wel_ref[2 * D:, :])
        c2e = (_dot(x, wel_ref[: 2 * D, :]) + _group_sum_roll(y, 2)
               + bel_ref[0:1, :])
        pre = c2e + sc_ref[0, 0] * rep_ref[...]
        h1 = _dot(pre, w1_ref[...])
        h1_ref[...] = h1.astype(odt)

        @pl.when(i == 0)
        def _():
            mom_ref[...] = jnp.zeros_like(mom_ref)

        mom_ref[0:1, :] += jnp.sum(h1, axis=0, keepdims=True)
        mom_ref[1:2, :] += jnp.sum(h1 * h1, axis=0, keepdims=True)

    return pl.pallas_call(
        body,
        grid=(nb,),
        in_specs=[
            _rows(rb, D),
            _rows(rb, D),
            _rows(rb, D),
            _full((4 * D, D)),
            _full((1, D)),
            _full((D, 2 * D)),
            pl.BlockSpec(memory_space=pltpu.SMEM),
        ],
        out_specs=[
            _rows(rb, 2 * D),
            _full((8, 2 * D)),
        ],
        out_shape=[
            jax.ShapeDtypeStruct((b, 2 * D), odt),
            jax.ShapeDtypeStruct((8, 2 * D), F32),
        ],
    )


@functools.lru_cache(maxsize=None)
def _make_mlp_mid(b, c1, c2, rb, idt=F32, odt=F32):
    nb = b // rb
    inv_n = 1.0 / b

    def body(h1_ref, m1_ref, g_ref, bb_ref, w2_ref, h2_ref, mom_ref):
        i = pl.program_id(0)
        mean = m1_ref[0:1, :] * inv_n
        var = m1_ref[1:2, :] * inv_n - mean * mean
        inv = lax.rsqrt(var + 1e-5)
        a = jnp.maximum(
            (h1_ref[...].astype(F32) - mean) * inv * g_ref[0:1, :]
            + bb_ref[0:1, :], 0.0)
        h2 = _dot(a, w2_ref[...])
        h2_ref[...] = h2.astype(odt)

        @pl.when(i == 0)
        def _():
            mom_ref[...] = jnp.zeros_like(mom_ref)

        mom_ref[0:1, :] += jnp.sum(h2, axis=0, keepdims=True)
        mom_ref[1:2, :] += jnp.sum(h2 * h2, axis=0, keepdims=True)

    return pl.pallas_call(
        body,
        grid=(nb,),
        in_specs=[
            _rows(rb, c1),
            _full((8, c1)),
            _full((1, c1)),
            _full((1, c1)),
            _full((c1, c2)),
        ],
        out_specs=[
            _rows(rb, c2),
            _full((8, c2)),
        ],
        out_shape=[
            jax.ShapeDtypeStruct((b, c2), odt),
            jax.ShapeDtypeStruct((8, c2), F32),
        ],
    )


@functools.lru_cache(maxsize=None)
def _make_mlp_tail(b, c2, rb, idt=F32):
    nb = b // rb
    inv_n = 1.0 / b

    def body(h2_ref, m2_ref, g_ref, bb_ref, o_ref):
        mean = m2_ref[0:1, :] * inv_n
        var = m2_ref[1:2, :] * inv_n - mean * mean
        inv = lax.rsqrt(var + 1e-5)
        o_ref[...] = jnp.maximum(
            (h2_ref[...].astype(F32) - mean) * inv * g_ref[0:1, :]
            + bb_ref[0:1, :], 0.0)

    return pl.pallas_call(
        body,
        grid=(nb,),
        in_specs=[
            _rows(rb, c2),
            _full((8, c2)),
            _full((1, c2)),
            _full((1, c2)),
        ],
        out_specs=_rows(rb, c2),
        out_shape=jax.ShapeDtypeStruct((b, c2), F32),
    )


def kernel(edge_rep, cycle5_rep, cycle6_rep, We1, ge1, be1, We2, ge2, be2,
           Wc1, gc1, bc1, Wc2, gc2, bc2, Wel, bel, Wl5, bl5, Wl6, bl6,
           edge_eps, cycle_eps, edge_nodes, cycle5_nodes, cycle6_nodes):
    be_, b5, b6 = edge_rep.shape[0], cycle5_rep.shape[0], cycle6_rep.shape[0]
    en = edge_nodes.astype(jnp.int32)
    c5n = cycle5_nodes.astype(jnp.int32)
    c6n = cycle6_nodes.astype(jnp.int32)
    zero = jnp.zeros((N, D), F32)
    e_sc = (1.0 + edge_eps).reshape(1, 1).astype(F32)
    c_sc = (1.0 + cycle_eps).reshape(1, 1).astype(F32)
    row1 = lambda v: v.reshape(1, -1).astype(F32)

    bf16 = jnp.bfloat16
    zbf = jnp.zeros((N, D), bf16)

    A = _make_segsum((be_,), D)(zero, edge_rep, en)
    A_rep = _make_segsum((b5, b6), D)(zero, cycle5_rep, c5n, cycle6_rep, c6n)

    x5a, x5b = _make_gather_x(b5, 5)(A, c5n)
    x6a, x6b = _make_gather_x(b6, 6)(A, c6n)

    A5a = _make_segsum((b5,), D)(zero, x5a, c5n)
    A5b = _make_segsum((b5,), D)(zero, x5b, c5n)
    A6a = _make_segsum((b6,), D)(zero, x6a, c6n)
    A6b = _make_segsum((b6,), D)(zero, x6b, c6n)
    l5a = _make_gather(b5, D)(A5a, c5n)
    l5b = _make_gather(b5, D)(A5b, c5n)
    l6a = _make_gather(b6, D)(A6a, c6n)
    l6b = _make_gather(b6, D)(A6b, c6n)

    Wl5f = Wl5.astype(F32)
    Wl6f = Wl6.astype(F32)
    loc_b = _make_gather(be_, D)(A_rep, en)
    e2c5, h1_5, m5 = _make_cycle_head(b5, 5, 2000)(
        l5a, l5b, cycle5_rep, Wl5f, row1(bl5), Wc1, c_sc)
    e2c6, h1_6, m6 = _make_cycle_head(b6, 6, 2400)(
        l6a, l6b, cycle6_rep, Wl6f, row1(bl6), Wc1, c_sc)

    A_e2c = _make_segsum((b5, b6), D)(zero, e2c5, c5n, e2c6, c6n)
    loc_a = _make_gather(be_, D)(A_e2c, en)

    h2_5, m52 = _make_mlp_mid(b5, 2 * D, D, 2000)(
        h1_5, m5, row1(gc1), row1(bc1), Wc2)
    c5_out = _make_mlp_tail(b5, D, 2000)(
        h2_5, m52, row1(gc2), row1(bc2))

    h2_6, m62 = _make_mlp_mid(b6, 2 * D, D, 2400)(
        h1_6, m6, row1(gc1), row1(bc1), Wc2)
    c6_out = _make_mlp_tail(b6, D, 2400)(
        h2_6, m62, row1(gc2), row1(bc2))

    h1_e, me = _make_edge_head(be_, 2560)(
        loc_a, loc_b, edge_rep, Wel.astype(F32), row1(bel), We1, e_sc)

    h2_e, me2 = _make_mlp_mid(be_, 2 * D, D, 2560)(
        h1_e, me, row1(ge1), row1(be1), We2)
    edge_out = _make_mlp_tail(be_, D, 2560)(
        h2_e, me2, row1(ge2), row1(be2))

    return (edge_out, c5_out, c6_out)


# speedup vs baseline: 1.0809x; 1.0809x over previous
import functools

import jax
import jax.numpy as jnp
from jax import lax
from jax.experimental import pallas as pl
from jax.experimental.pallas import tpu as pltpu
from jax.experimental.pallas import tpu_sc as plsc

N = 10000
D = 128
F32 = jnp.float32
CHUNK = 128


def _sc_mesh():
    return plsc.VectorSubcoreMesh(core_axis_name="c", subcore_axis_name="s")


def _on_slot(pt, fn):
    @pl.when(pt == 0)
    def _():
        fn(0)

    @pl.when(pt == 1)
    def _():
        fn(1)


_SC_PARAMS = pltpu.CompilerParams(use_tc_tiling_on_sc=False)


@functools.lru_cache(maxsize=None)
def _make_segsum(bs, d, dt=F32):
    dh = d // 2
    npr = N // 16
    tails = [b % CHUNK for b in bs]
    scratch = [
        pltpu.VMEM_SHARED((N, dh), dt),
        pltpu.VMEM((CHUNK, dh), dt),
        pltpu.VMEM((CHUNK, dh), dt),
        pltpu.VMEM((CHUNK,), jnp.int32),
        pltpu.VMEM((CHUNK,), jnp.int32),
        pltpu.SemaphoreType.DMA,
        pltpu.SemaphoreType.DMA,
        pltpu.SemaphoreType.DMA,
    ]
    for t in tails:
        if t:
            scratch.append(pltpu.VMEM((t, dh), dt))
            scratch.append(pltpu.VMEM((t,), jnp.int32))

    @functools.partial(
        pl.kernel,
        out_type=jax.ShapeDtypeStruct((N, d), dt),
        mesh=_sc_mesh(),
        scratch_types=scratch,
        compiler_params=_SC_PARAMS,
    )
    def seg(*refs):
        zero_hbm = refs[0]
        data = refs[1:1 + 2 * len(bs)]
        out = refs[1 + 2 * len(bs)]
        rest = list(refs[2 + 2 * len(bs):])
        acc = rest[0]
        rbuf = rest[1:3]
        ibuf = rest[3:5]
        lsem = rest[5]
        ssem = rest[6:8]
        tbufs = rest[8:]
        c = lax.axis_index("c")
        s = lax.axis_index("s")
        pltpu.sync_copy(zero_hbm.at[pl.ds(s * npr, npr), pl.ds(0, dh)],
                        acc.at[pl.ds(s * npr, npr)])
        plsc.subcore_barrier()
        ti = 0
        for j, b in enumerate(bs):
            rows_hbm = data[2 * j]
            idx_hbm = data[2 * j + 1]
            nch = b // CHUNK
            tail = tails[j]

            def start_loads(i, p, rows_hbm=rows_hbm, idx_hbm=idx_hbm):
                base = i * CHUNK
                pltpu.async_copy(
                    rows_hbm.at[pl.ds(base, CHUNK), pl.ds(c * dh, dh)],
                    rbuf[p], lsem)
                pltpu.async_copy(idx_hbm.at[pl.ds(base, CHUNK)], ibuf[p],
                                 lsem)

            def wait_loads(p):
                pltpu.make_async_copy(
                    rows_hbm.at[pl.ds(0, CHUNK), pl.ds(0, dh)],
                    rbuf[p], lsem).wait()
                pltpu.make_async_copy(idx_hbm.at[pl.ds(0, CHUNK)], ibuf[p],
                                      lsem).wait()

            def wait_scat(p):
                pltpu.make_async_copy(rbuf[p], acc.at[ibuf[p]],
                                      ssem[p]).wait()

            nj = jnp.maximum(0, (nch - s + 15) // 16)

            @pl.when(nj > 0)
            def _():
                start_loads(s, 0)

            @pl.loop(0, nj)
            def _(jj):
                p = lax.rem(jj, 2)
                _on_slot(p, wait_loads)
                _on_slot(p, lambda q: pltpu.async_copy(
                    rbuf[q], acc.at[ibuf[q]], ssem[q], add=True))

                @pl.when(jj + 1 < nj)
                def _():
                    @pl.when(jj >= 1)
                    def _():
                        _on_slot(1 - p, wait_scat)

                    _on_slot(1 - p,
                             lambda q: start_loads(s + (jj + 1) * 16, q))

            @pl.when(nj >= 2)
            def _():
                _on_slot(lax.rem(nj, 2), wait_scat)

            @pl.when(nj >= 1)
            def _():
                _on_slot(lax.rem(nj + 1, 2), wait_scat)

            if tail:
                trbuf, tibuf = tbufs[ti], tbufs[ti + 1]
                ti += 2

                @pl.when(s == nch % 16)
                def _():
                    base = nch * CHUNK
                    pltpu.sync_copy(
                        rows_hbm.at[pl.ds(base, tail), pl.ds(c * dh, dh)],
                        trbuf)
                    pltpu.sync_copy(idx_hbm.at[pl.ds(base, tail)], tibuf)
                    pltpu.sync_copy(trbuf, acc.at[tibuf], add=True)
        plsc.subcore_barrier()
        pltpu.sync_copy(acc.at[pl.ds(s * npr, npr)],
                        out.at[pl.ds(s * npr, npr), pl.ds(c * dh, dh)])

    return seg


@functools.lru_cache(maxsize=None)
def _make_gather(b, d, dt=F32):
    nch = b // CHUNK
    tail = b % CHUNK
    scratch = [
        pltpu.VMEM((CHUNK, d), dt),
        pltpu.VMEM((CHUNK, d), dt),
        pltpu.VMEM((CHUNK,), jnp.int32),
        pltpu.VMEM((CHUNK,), jnp.int32),
        pltpu.SemaphoreType.DMA,
        pltpu.SemaphoreType.DMA,
    ]
    if tail:
        scratch.append(pltpu.VMEM((tail, d), dt))
        scratch.append(pltpu.VMEM((tail,), jnp.int32))

    @functools.partial(
        pl.kernel,
        out_type=jax.ShapeDtypeStruct((b, d), dt),
        mesh=_sc_mesh(),
        scratch_types=scratch,
        compiler_params=_SC_PARAMS,
    )
    def gat(*refs):
        table_hbm, idx_hbm, out_hbm = refs[:3]
        rest = list(refs[3:])
        rbuf = rest[0:2]
        ibuf = rest[2:4]
        gsem = rest[4:6]
        c = lax.axis_index("c")
        s = lax.axis_index("s")
        w = s * 2 + c
        nj = jnp.maximum(0, (nch - w + 31) // 32)

        def load_idx(t, q):
            pltpu.sync_copy(idx_hbm.at[pl.ds((w + 32 * t) * CHUNK, CHUNK)],
                            ibuf[q])

        def start_gather(q):
            pltpu.async_copy(table_hbm.at[ibuf[q]], rbuf[q], gsem[q])

        def wait_gather(q):
            pltpu.make_async_copy(table_hbm.at[ibuf[q]], rbuf[q],
                                  gsem[q]).wait()

        @pl.when(nj > 0)
        def _():
            load_idx(0, 0)
            start_gather(0)

        @pl.loop(0, nj)
        def _(jj):
            p = lax.rem(jj, 2)

            @pl.when(jj + 1 < nj)
            def _():
                _on_slot(1 - p, lambda q: load_idx(jj + 1, q))

            _on_slot(p, wait_gather)

            @pl.when(jj + 1 < nj)
            def _():
                _on_slot(1 - p, start_gather)

            _on_slot(p, lambda q: pltpu.sync_copy(
                rbuf[q],
                out_hbm.at[pl.ds((w + 32 * jj) * CHUNK, CHUNK)]))

        if tail:
            trbuf, tibuf = rest[6], rest[7]

            @pl.when(w == nch % 32)
            def _():
                base = nch * CHUNK
                pltpu.sync_copy(idx_hbm.at[pl.ds(base, tail)], tibuf)
                pltpu.async_copy(table_hbm.at[tibuf], trbuf, gsem[0]).wait()
                pltpu.sync_copy(trbuf, out_hbm.at[pl.ds(base, tail)])

    return gat


GCH = 120


@functools.lru_cache(maxsize=None)
def _make_gather_x(b, k):
    nch = b // GCH
    tail = b % GCH
    assert tail % k == 0
    scratch = [
        pltpu.VMEM((GCH, D), F32),
        pltpu.VMEM((GCH, D), F32),
        pltpu.VMEM((GCH, D), F32),
        pltpu.VMEM((GCH, D), F32),
        pltpu.VMEM((GCH,), jnp.int32),
        pltpu.VMEM((GCH,), jnp.int32),
        pltpu.SemaphoreType.DMA,
        pltpu.SemaphoreType.DMA,
    ]

    @functools.partial(
        pl.kernel,
        out_type=(jax.ShapeDtypeStruct((b, D), F32),
                  jax.ShapeDtypeStruct((b, D), F32)),
        mesh=_sc_mesh(),
        scratch_types=scratch,
        compiler_params=_SC_PARAMS,
    )
    def gatx(*refs):
        table_hbm, idx_hbm, oloc_hbm, obc_hbm = refs[:4]
        rest = list(refs[4:])
        rbuf = rest[0:2]
        bbuf = rest[2:4]
        ibuf = rest[4:6]
        gsem = rest[6:8]
        c = lax.axis_index("c")
        s = lax.axis_index("s")
        w = s * 2 + c
        nj = jnp.maximum(0, (nch - w + 31) // 32)

        def load_idx(t, q):
            pltpu.sync_copy(idx_hbm.at[pl.ds((w + 32 * t) * GCH, GCH)],
                            ibuf[q])

        def start_gather(q):
            pltpu.async_copy(table_hbm.at[ibuf[q]], rbuf[q], gsem[q])

        def wait_gather(q):
            pltpu.make_async_copy(table_hbm.at[ibuf[q]], rbuf[q],
                                  gsem[q]).wait()

        def bcast_compute(q, nrows):
            for g in range(nrows // k):
                for c8 in range(D // 16):
                    sl = pl.ds(c8 * 16, 16)
                    acc = rbuf[q][g * k, sl]
                    for r in range(1, k):
                        acc = acc + rbuf[q][g * k + r, sl]
                    for r in range(k):
                        bbuf[q][g * k + r, sl] = acc

        def write_out(q, t):
            base = (w + 32 * t) * GCH
            pltpu.sync_copy(rbuf[q], oloc_hbm.at[pl.ds(base, GCH)])
            pltpu.sync_copy(bbuf[q], obc_hbm.at[pl.ds(base, GCH)])

        @pl.when(nj > 0)
        def _():
            load_idx(0, 0)
            start_gather(0)

        @pl.loop(0, nj)
        def _(jj):
            p = lax.rem(jj, 2)

            @pl.when(jj + 1 < nj)
            def _():
                _on_slot(1 - p, lambda q: load_idx(jj + 1, q))

            _on_slot(p, wait_gather)

            @pl.when(jj + 1 < nj)
            def _():
                _on_slot(1 - p, start_gather)

            _on_slot(p, lambda q: bcast_compute(q, GCH))
            _on_slot(p, lambda q: write_out(q, jj))

        if tail:

            @pl.when(w == nch % 32)
            def _():
                base = nch * GCH
                pltpu.sync_copy(idx_hbm.at[pl.ds(base, tail)],
                                ibuf[0].at[pl.ds(0, tail)])
                pltpu.async_copy(
                    table_hbm.at[ibuf[0].at[pl.ds(0, tail)]],
                    rbuf[0].at[pl.ds(0, tail)], gsem[0]).wait()
                bcast_compute(0, tail)
                pltpu.sync_copy(rbuf[0].at[pl.ds(0, tail)],
                                oloc_hbm.at[pl.ds(base, tail)])
                pltpu.sync_copy(bbuf[0].at[pl.ds(0, tail)],
                                obc_hbm.at[pl.ds(base, tail)])

    return gatx


def _group_bcast(x, k):
    r, ch = x.shape
    g = jnp.sum(x.reshape(r // k, k, ch), axis=1, keepdims=True)
    return jnp.broadcast_to(g, (r // k, k, ch)).reshape(r, ch)


def _group_sum_roll(y, k):
    r, ch = y.shape
    pos = lax.rem(lax.broadcasted_iota(jnp.int32, (r, ch), 0), k)
    acc = y
    for d in range(1, k):
        up = pltpu.roll(y, r - d, 0)
        down = pltpu.roll(y, d, 0)
        acc = acc + jnp.where(pos < k - d, up, 0.0)
        acc = acc + jnp.where(pos >= d, down, 0.0)
    return acc


def _full(shape):
    nd = len(shape)
    return pl.BlockSpec(shape, lambda i, _n=nd: (0,) * _n)


def _rows(rb, ch):
    return pl.BlockSpec((rb, ch), lambda i: (i, 0))


def _dot(a, b):
    return jnp.dot(a.astype(jnp.bfloat16), b.astype(jnp.bfloat16),
                   preferred_element_type=F32)


@functools.lru_cache(maxsize=None)
def _make_xbuild(b, k, rb, odt=F32):
    nb = b // rb

    def body(x_ref, o_ref):
        x = x_ref[...]
        o_ref[...] = jnp.concatenate(
            [x, _group_sum_roll(x, k)], axis=1).astype(odt)

    return pl.pallas_call(
        body,
        grid=(nb,),
        in_specs=[_rows(rb, D)],
        out_specs=_rows(rb, 2 * D),
        out_shape=jax.ShapeDtypeStruct((b, 2 * D), odt),
    )


@functools.lru_cache(maxsize=None)
def _make_cycle_head(b, k, rb, idt=F32, odt=F32, h1dt=F32):
    nb = b // rb

    def body(xa_ref, xb_ref, rep_ref, wl_ref, bl_ref, w1_ref, sc_ref,
             e2c_ref, h1_ref, mom_ref):
        i = pl.program_id(0)
        x32 = jnp.concatenate([xa_ref[...], xb_ref[...]],
                              axis=1).astype(F32)
        y = _dot(x32, wl_ref[2 * D:, :])
        e2c = (_dot(x32, wl_ref[: 2 * D, :]) + _group_sum_roll(y, k)
               + bl_ref[0:1, :])
        e2c_ref[...] = e2c.astype(odt)
        pre = e2c + sc_ref[0, 0] * rep_ref[...]
        h1 = _dot(pre, w1_ref[...])
        h1_ref[...] = h1.astype(h1dt)

        @pl.when(i == 0)
        def _():
            mom_ref[...] = jnp.zeros_like(mom_ref)

        mom_ref[0:1, :] += jnp.sum(h1, axis=0, keepdims=True)
        mom_ref[1:2, :] += jnp.sum(h1 * h1, axis=0, keepdims=True)

    return pl.pallas_call(
        body,
        grid=(nb,),
        in_specs=[
            _rows(rb, D),
            _rows(rb, D),
            _rows(rb, D),
            _full((4 * D, D)),
            _full((1, D)),
            _full((D, 2 * D)),
            pl.BlockSpec(memory_space=pltpu.SMEM),
        ],
        out_specs=[
            _rows(rb, D),
            _rows(rb, 2 * D),
            _full((8, 2 * D)),
        ],
        out_shape=[
            jax.ShapeDtypeStruct((b, D), odt),
            jax.ShapeDtypeStruct((b, 2 * D), h1dt),
            jax.ShapeDtypeStruct((8, 2 * D), F32),
        ],
    )


@functools.lru_cache(maxsize=None)
def _make_edge_head(b, rb, idt=F32, odt=F32):
    nb = b // rb

    def body(la_ref, lb_ref, rep_ref, wel_ref, bel_ref, w1_ref, sc_ref,
             h1_ref, mom_ref):
        i = pl.program_id(0)
        x = jnp.concatenate([la_ref[...], lb_ref[...]], axis=1).astype(F32)
        y = _dot(x, wel_ref[2 * D:, :])
        c2e = (_dot(x, wel_ref[: 2 * D, :]) + _group_sum_roll(y, 2)
               + bel_ref[0:1, :])
        pre = c2e + sc_ref[0, 0] * rep_ref[...]
        h1 = _dot(pre, w1_ref[...])
        h1_ref[...] = h1.astype(odt)

        @pl.when(i == 0)
        def _():
            mom_ref[...] = jnp.zeros_like(mom_ref)

        mom_ref[0:1, :] += jnp.sum(h1, axis=0, keepdims=True)
        mom_ref[1:2, :] += jnp.sum(h1 * h1, axis=0, keepdims=True)

    return pl.pallas_call(
        body,
        grid=(nb,),
        in_specs=[
            _rows(rb, D),
            _rows(rb, D),
            _rows(rb, D),
            _full((4 * D, D)),
            _full((1, D)),
            _full((D, 2 * D)),
            pl.BlockSpec(memory_space=pltpu.SMEM),
        ],
        out_specs=[
            _rows(rb, 2 * D),
            _full((8, 2 * D)),
        ],
        out_shape=[
            jax.ShapeDtypeStruct((b, 2 * D), odt),
            jax.ShapeDtypeStruct((8, 2 * D), F32),
        ],
    )


@functools.lru_cache(maxsize=None)
def _make_mlp_mid(b, c1, c2, rb, idt=F32, odt=F32):
    nb = b // rb
    inv_n = 1.0 / b

    def body(h1_ref, m1_ref, g_ref, bb_ref, w2_ref, h2_ref, mom_ref):
        i = pl.program_id(0)
        mean = m1_ref[0:1, :] * inv_n
        var = m1_ref[1:2, :] * inv_n - mean * mean
        inv = lax.rsqrt(var + 1e-5)
        a = jnp.maximum(
            (h1_ref[...].astype(F32) - mean) * inv * g_ref[0:1, :]
            + bb_ref[0:1, :], 0.0)
        h2 = _dot(a, w2_ref[...])
        h2_ref[...] = h2.astype(odt)

        @pl.when(i == 0)
        def _():
            mom_ref[...] = jnp.zeros_like(mom_ref)

        mom_ref[0:1, :] += jnp.sum(h2, axis=0, keepdims=True)
        mom_ref[1:2, :] += jnp.sum(h2 * h2, axis=0, keepdims=True)

    return pl.pallas_call(
        body,
        grid=(nb,),
        in_specs=[
            _rows(rb, c1),
            _full((8, c1)),
            _full((1, c1)),
            _full((1, c1)),
            _full((c1, c2)),
        ],
        out_specs=[
            _rows(rb, c2),
            _full((8, c2)),
        ],
        out_shape=[
            jax.ShapeDtypeStruct((b, c2), odt),
            jax.ShapeDtypeStruct((8, c2), F32),
        ],
    )


@functools.lru_cache(maxsize=None)
def _make_mlp_tail(b, c2, rb, idt=F32):
    nb = b // rb
    inv_n = 1.0 / b

    def body(h2_ref, m2_ref, g_ref, bb_ref, o_ref):
        mean = m2_ref[0:1, :] * inv_n
        var = m2_ref[1:2, :] * inv_n - mean * mean
        inv = lax.rsqrt(var + 1e-5)
        o_ref[...] = jnp.maximum(
            (h2_ref[...].astype(F32) - mean) * inv * g_ref[0:1, :]
            + bb_ref[0:1, :], 0.0)

    return pl.pallas_call(
        body,
        grid=(nb,),
        in_specs=[
            _rows(rb, c2),
            _full((8, c2)),
            _full((1, c2)),
            _full((1, c2)),
        ],
        out_specs=_rows(rb, c2),
        out_shape=jax.ShapeDtypeStruct((b, c2), F32),
    )


def kernel(edge_rep, cycle5_rep, cycle6_rep, We1, ge1, be1, We2, ge2, be2,
           Wc1, gc1, bc1, Wc2, gc2, bc2, Wel, bel, Wl5, bl5, Wl6, bl6,
           edge_eps, cycle_eps, edge_nodes, cycle5_nodes, cycle6_nodes):
    be_, b5, b6 = edge_rep.shape[0], cycle5_rep.shape[0], cycle6_rep.shape[0]
    en = edge_nodes.astype(jnp.int32)
    c5n = cycle5_nodes.astype(jnp.int32)
    c6n = cycle6_nodes.astype(jnp.int32)
    zero = jnp.zeros((N, D), F32)
    e_sc = (1.0 + edge_eps).reshape(1, 1).astype(F32)
    c_sc = (1.0 + cycle_eps).reshape(1, 1).astype(F32)
    row1 = lambda v: v.reshape(1, -1).astype(F32)

    bf16 = jnp.bfloat16
    zbf = jnp.zeros((N, D), bf16)

    A = _make_segsum((be_,), D)(zero, edge_rep, en)
    A_rep = _make_segsum((b5, b6), D)(zero, cycle5_rep, c5n, cycle6_rep, c6n)

    x5a, x5b = _make_gather_x(b5, 5)(A, c5n)
    x6a, x6b = _make_gather_x(b6, 6)(A, c6n)

    A5a = _make_segsum((b5,), D)(zero, x5a, c5n)
    A5b = _make_segsum((b5,), D)(zero, x5b, c5n)
    A6a = _make_segsum((b6,), D)(zero, x6a, c6n)
    A6b = _make_segsum((b6,), D)(zero, x6b, c6n)
    l5a = _make_gather(b5, D)(A5a, c5n)
    l5b = _make_gather(b5, D)(A5b, c5n)
    l6a = _make_gather(b6, D)(A6a, c6n)
    l6b = _make_gather(b6, D)(A6b, c6n)

    Wl5f = Wl5.astype(F32)
    Wl6f = Wl6.astype(F32)
    loc_b = _make_gather(be_, D)(A_rep, en)
    bf16 = jnp.bfloat16
    e2c5, h1_5, m5 = _make_cycle_head(b5, 5, 2000, F32, F32, bf16)(
        l5a, l5b, cycle5_rep, Wl5f, row1(bl5), Wc1, c_sc)
    e2c6, h1_6, m6 = _make_cycle_head(b6, 6, 2400, F32, F32, bf16)(
        l6a, l6b, cycle6_rep, Wl6f, row1(bl6), Wc1, c_sc)

    A_e2c = _make_segsum((b5, b6), D)(zero, e2c5, c5n, e2c6, c6n)
    loc_a = _make_gather(be_, D)(A_e2c, en)

    h2_5, m52 = _make_mlp_mid(b5, 2 * D, D, 2000, bf16, bf16)(
        h1_5, m5, row1(gc1), row1(bc1), Wc2)
    c5_out = _make_mlp_tail(b5, D, 2000, bf16)(
        h2_5, m52, row1(gc2), row1(bc2))

    h2_6, m62 = _make_mlp_mid(b6, 2 * D, D, 2400, bf16, bf16)(
        h1_6, m6, row1(gc1), row1(bc1), Wc2)
    c6_out = _make_mlp_tail(b6, D, 2400, bf16)(
        h2_6, m62, row1(gc2), row1(bc2))

    h1_e, me = _make_edge_head(be_, 2560, F32, bf16)(
        loc_a, loc_b, edge_rep, Wel.astype(F32), row1(bel), We1, e_sc)

    h2_e, me2 = _make_mlp_mid(be_, 2 * D, D, 2560, bf16, bf16)(
        h1_e, me, row1(ge1), row1(be1), We2)
    edge_out = _make_mlp_tail(be_, D, 2560, bf16)(
        h2_e, me2, row1(ge2), row1(be2))

    return (edge_out, c5_out, c6_out)


# speedup vs baseline: 1.1384x; 1.0532x over previous
import functools

import jax
import jax.numpy as jnp
from jax import lax
from jax.experimental import pallas as pl
from jax.experimental.pallas import tpu as pltpu
from jax.experimental.pallas import tpu_sc as plsc

N = 10000
D = 128
F32 = jnp.float32
CHUNK = 128


def _sc_mesh():
    return plsc.VectorSubcoreMesh(core_axis_name="c", subcore_axis_name="s")


def _on_slot(pt, fn):
    @pl.when(pt == 0)
    def _():
        fn(0)

    @pl.when(pt == 1)
    def _():
        fn(1)


_SC_PARAMS = pltpu.CompilerParams(use_tc_tiling_on_sc=False)


@functools.lru_cache(maxsize=None)
def _make_segsum(bs, d, dt=F32):
    dh = d // 2
    npr = N // 16
    tails = [b % CHUNK for b in bs]
    scratch = [
        pltpu.VMEM_SHARED((N, dh), dt),
        pltpu.VMEM((CHUNK, dh), dt),
        pltpu.VMEM((CHUNK, dh), dt),
        pltpu.VMEM((CHUNK,), jnp.int32),
        pltpu.VMEM((CHUNK,), jnp.int32),
        pltpu.SemaphoreType.DMA,
        pltpu.SemaphoreType.DMA,
        pltpu.SemaphoreType.DMA,
    ]
    for t in tails:
        if t:
            scratch.append(pltpu.VMEM((t, dh), dt))
            scratch.append(pltpu.VMEM((t,), jnp.int32))

    @functools.partial(
        pl.kernel,
        out_type=jax.ShapeDtypeStruct((N, d), dt),
        mesh=_sc_mesh(),
        scratch_types=scratch,
        compiler_params=_SC_PARAMS,
    )
    def seg(*refs):
        zero_hbm = refs[0]
        data = refs[1:1 + 2 * len(bs)]
        out = refs[1 + 2 * len(bs)]
        rest = list(refs[2 + 2 * len(bs):])
        acc = rest[0]
        rbuf = rest[1:3]
        ibuf = rest[3:5]
        lsem = rest[5]
        ssem = rest[6:8]
        tbufs = rest[8:]
        c = lax.axis_index("c")
        s = lax.axis_index("s")
        pltpu.sync_copy(zero_hbm.at[pl.ds(s * npr, npr), pl.ds(0, dh)],
                        acc.at[pl.ds(s * npr, npr)])
        plsc.subcore_barrier()
        ti = 0
        for j, b in enumerate(bs):
            rows_hbm = data[2 * j]
            idx_hbm = data[2 * j + 1]
            nch = b // CHUNK
            tail = tails[j]

            def start_loads(i, p, rows_hbm=rows_hbm, idx_hbm=idx_hbm):
                base = i * CHUNK
                pltpu.async_copy(
                    rows_hbm.at[pl.ds(base, CHUNK), pl.ds(c * dh, dh)],
                    rbuf[p], lsem)
                pltpu.async_copy(idx_hbm.at[pl.ds(base, CHUNK)], ibuf[p],
                                 lsem)

            def wait_loads(p):
                pltpu.make_async_copy(
                    rows_hbm.at[pl.ds(0, CHUNK), pl.ds(0, dh)],
                    rbuf[p], lsem).wait()
                pltpu.make_async_copy(idx_hbm.at[pl.ds(0, CHUNK)], ibuf[p],
                                      lsem).wait()

            def wait_scat(p):
                pltpu.make_async_copy(rbuf[p], acc.at[ibuf[p]],
                                      ssem[p]).wait()

            nj = jnp.maximum(0, (nch - s + 15) // 16)

            @pl.when(nj > 0)
            def _():
                start_loads(s, 0)

            @pl.loop(0, nj)
            def _(jj):
                p = lax.rem(jj, 2)
                _on_slot(p, wait_loads)
                _on_slot(p, lambda q: pltpu.async_copy(
                    rbuf[q], acc.at[ibuf[q]], ssem[q], add=True))

                @pl.when(jj + 1 < nj)
                def _():
                    @pl.when(jj >= 1)
                    def _():
                        _on_slot(1 - p, wait_scat)

                    _on_slot(1 - p,
                             lambda q: start_loads(s + (jj + 1) * 16, q))

            @pl.when(nj >= 2)
            def _():
                _on_slot(lax.rem(nj, 2), wait_scat)

            @pl.when(nj >= 1)
            def _():
                _on_slot(lax.rem(nj + 1, 2), wait_scat)

            if tail:
                trbuf, tibuf = tbufs[ti], tbufs[ti + 1]
                ti += 2

                @pl.when(s == nch % 16)
                def _():
                    base = nch * CHUNK
                    pltpu.sync_copy(
                        rows_hbm.at[pl.ds(base, tail), pl.ds(c * dh, dh)],
                        trbuf)
                    pltpu.sync_copy(idx_hbm.at[pl.ds(base, tail)], tibuf)
                    pltpu.sync_copy(trbuf, acc.at[tibuf], add=True)
        plsc.subcore_barrier()
        pltpu.sync_copy(acc.at[pl.ds(s * npr, npr)],
                        out.at[pl.ds(s * npr, npr), pl.ds(c * dh, dh)])

    return seg


@functools.lru_cache(maxsize=None)
def _make_gather(b, d, dt=F32):
    nch = b // CHUNK
    tail = b % CHUNK
    scratch = [
        pltpu.VMEM((CHUNK, d), dt),
        pltpu.VMEM((CHUNK, d), dt),
        pltpu.VMEM((CHUNK,), jnp.int32),
        pltpu.VMEM((CHUNK,), jnp.int32),
        pltpu.SemaphoreType.DMA,
        pltpu.SemaphoreType.DMA,
    ]
    if tail:
        scratch.append(pltpu.VMEM((tail, d), dt))
        scratch.append(pltpu.VMEM((tail,), jnp.int32))

    @functools.partial(
        pl.kernel,
        out_type=jax.ShapeDtypeStruct((b, d), dt),
        mesh=_sc_mesh(),
        scratch_types=scratch,
        compiler_params=_SC_PARAMS,
    )
    def gat(*refs):
        table_hbm, idx_hbm, out_hbm = refs[:3]
        rest = list(refs[3:])
        rbuf = rest[0:2]
        ibuf = rest[2:4]
        gsem = rest[4:6]
        c = lax.axis_index("c")
        s = lax.axis_index("s")
        w = s * 2 + c
        nj = jnp.maximum(0, (nch - w + 31) // 32)

        def load_idx(t, q):
            pltpu.sync_copy(idx_hbm.at[pl.ds((w + 32 * t) * CHUNK, CHUNK)],
                            ibuf[q])

        def start_gather(q):
            pltpu.async_copy(table_hbm.at[ibuf[q]], rbuf[q], gsem[q])

        def wait_gather(q):
            pltpu.make_async_copy(table_hbm.at[ibuf[q]], rbuf[q],
                                  gsem[q]).wait()

        @pl.when(nj > 0)
        def _():
            load_idx(0, 0)
            start_gather(0)

        @pl.loop(0, nj)
        def _(jj):
            p = lax.rem(jj, 2)

            @pl.when(jj + 1 < nj)
            def _():
                _on_slot(1 - p, lambda q: load_idx(jj + 1, q))

            _on_slot(p, wait_gather)

            @pl.when(jj + 1 < nj)
            def _():
                _on_slot(1 - p, start_gather)

            _on_slot(p, lambda q: pltpu.sync_copy(
                rbuf[q],
                out_hbm.at[pl.ds((w + 32 * jj) * CHUNK, CHUNK)]))

        if tail:
            trbuf, tibuf = rest[6], rest[7]

            @pl.when(w == nch % 32)
            def _():
                base = nch * CHUNK
                pltpu.sync_copy(idx_hbm.at[pl.ds(base, tail)], tibuf)
                pltpu.async_copy(table_hbm.at[tibuf], trbuf, gsem[0]).wait()
                pltpu.sync_copy(trbuf, out_hbm.at[pl.ds(base, tail)])

    return gat


GCH = 120


@functools.lru_cache(maxsize=None)
def _make_gather_x(b, k):
    nch = b // GCH
    tail = b % GCH
    assert tail % k == 0
    scratch = [
        pltpu.VMEM((GCH, D), F32),
        pltpu.VMEM((GCH, D), F32),
        pltpu.VMEM((GCH, D), F32),
        pltpu.VMEM((GCH, D), F32),
        pltpu.VMEM((GCH,), jnp.int32),
        pltpu.VMEM((GCH,), jnp.int32),
        pltpu.SemaphoreType.DMA,
        pltpu.SemaphoreType.DMA,
    ]

    @functools.partial(
        pl.kernel,
        out_type=(jax.ShapeDtypeStruct((b, D), F32),
                  jax.ShapeDtypeStruct((b, D), F32)),
        mesh=_sc_mesh(),
        scratch_types=scratch,
        compiler_params=_SC_PARAMS,
    )
    def gatx(*refs):
        table_hbm, idx_hbm, oloc_hbm, obc_hbm = refs[:4]
        rest = list(refs[4:])
        rbuf = rest[0:2]
        bbuf = rest[2:4]
        ibuf = rest[4:6]
        gsem = rest[6:8]
        c = lax.axis_index("c")
        s = lax.axis_index("s")
        w = s * 2 + c
        nj = jnp.maximum(0, (nch - w + 31) // 32)

        def load_idx(t, q):
            pltpu.sync_copy(idx_hbm.at[pl.ds((w + 32 * t) * GCH, GCH)],
                            ibuf[q])

        def start_gather(q):
            pltpu.async_copy(table_hbm.at[ibuf[q]], rbuf[q], gsem[q])

        def wait_gather(q):
            pltpu.make_async_copy(table_hbm.at[ibuf[q]], rbuf[q],
                                  gsem[q]).wait()

        def bcast_compute(q, nrows):
            for g in range(nrows // k):
                for c8 in range(D // 16):
                    sl = pl.ds(c8 * 16, 16)
                    acc = rbuf[q][g * k, sl]
                    for r in range(1, k):
                        acc = acc + rbuf[q][g * k + r, sl]
                    for r in range(k):
                        bbuf[q][g * k + r, sl] = acc

        def write_out(q, t):
            base = (w + 32 * t) * GCH
            pltpu.sync_copy(rbuf[q], oloc_hbm.at[pl.ds(base, GCH)])
            pltpu.sync_copy(bbuf[q], obc_hbm.at[pl.ds(base, GCH)])

        @pl.when(nj > 0)
        def _():
            load_idx(0, 0)
            start_gather(0)

        @pl.loop(0, nj)
        def _(jj):
            p = lax.rem(jj, 2)

            @pl.when(jj + 1 < nj)
            def _():
                _on_slot(1 - p, lambda q: load_idx(jj + 1, q))

            _on_slot(p, wait_gather)

            @pl.when(jj + 1 < nj)
            def _():
                _on_slot(1 - p, start_gather)

            _on_slot(p, lambda q: bcast_compute(q, GCH))
            _on_slot(p, lambda q: write_out(q, jj))

        if tail:

            @pl.when(w == nch % 32)
            def _():
                base = nch * GCH
                pltpu.sync_copy(idx_hbm.at[pl.ds(base, tail)],
                                ibuf[0].at[pl.ds(0, tail)])
                pltpu.async_copy(
                    table_hbm.at[ibuf[0].at[pl.ds(0, tail)]],
                    rbuf[0].at[pl.ds(0, tail)], gsem[0]).wait()
                bcast_compute(0, tail)
                pltpu.sync_copy(rbuf[0].at[pl.ds(0, tail)],
                                oloc_hbm.at[pl.ds(base, tail)])
                pltpu.sync_copy(bbuf[0].at[pl.ds(0, tail)],
                                obc_hbm.at[pl.ds(base, tail)])

    return gatx


def _group_bcast(x, k):
    r, ch = x.shape
    g = jnp.sum(x.reshape(r // k, k, ch), axis=1, keepdims=True)
    return jnp.broadcast_to(g, (r // k, k, ch)).reshape(r, ch)


def _group_sum_roll(y, k):
    r, ch = y.shape
    pos = lax.rem(lax.broadcasted_iota(jnp.int32, (r, ch), 0), k)
    acc = y
    for d in range(1, k):
        up = pltpu.roll(y, r - d, 0)
        down = pltpu.roll(y, d, 0)
        acc = acc + jnp.where(pos < k - d, up, 0.0)
        acc = acc + jnp.where(pos >= d, down, 0.0)
    return acc


def _full(shape):
    nd = len(shape)
    return pl.BlockSpec(shape, lambda i, _n=nd: (0,) * _n)


def _rows(rb, ch):
    return pl.BlockSpec((rb, ch), lambda i: (i, 0))


def _dot(a, b):
    return jnp.dot(a.astype(jnp.bfloat16), b.astype(jnp.bfloat16),
                   preferred_element_type=F32)


@functools.lru_cache(maxsize=None)
def _make_xbuild(b, k, rb, odt=F32):
    nb = b // rb

    def body(x_ref, o_ref):
        x = x_ref[...]
        o_ref[...] = jnp.concatenate(
            [x, _group_sum_roll(x, k)], axis=1).astype(odt)

    return pl.pallas_call(
        body,
        grid=(nb,),
        in_specs=[_rows(rb, D)],
        out_specs=_rows(rb, 2 * D),
        out_shape=jax.ShapeDtypeStruct((b, 2 * D), odt),
    )


@functools.lru_cache(maxsize=None)
def _make_cycle_head(b, k, rb, idt=F32, odt=F32, h1dt=F32):
    nb = b // rb

    def body(xa_ref, xb_ref, rep_ref, wl_ref, bl_ref, w1_ref, sc_ref,
             e2c_ref, h1_ref, mom_ref):
        i = pl.program_id(0)
        x32 = jnp.concatenate([xa_ref[...], xb_ref[...]],
                              axis=1).astype(F32)
        y = _dot(x32, wl_ref[2 * D:, :])
        e2c = (_dot(x32, wl_ref[: 2 * D, :]) + _group_sum_roll(y, k)
               + bl_ref[0:1, :])
        e2c_ref[...] = e2c.astype(odt)
        pre = e2c + sc_ref[0, 0] * rep_ref[...]
        h1 = _dot(pre, w1_ref[...])
        h1_ref[...] = h1.astype(h1dt)

        @pl.when(i == 0)
        def _():
            mom_ref[...] = jnp.zeros_like(mom_ref)

        mom_ref[0:1, :] += jnp.sum(h1, axis=0, keepdims=True)
        mom_ref[1:2, :] += jnp.sum(h1 * h1, axis=0, keepdims=True)

    return pl.pallas_call(
        body,
        grid=(nb,),
        in_specs=[
            _rows(rb, D),
            _rows(rb, D),
            _rows(rb, D),
            _full((4 * D, D)),
            _full((1, D)),
            _full((D, 2 * D)),
            pl.BlockSpec(memory_space=pltpu.SMEM),
        ],
        out_specs=[
            _rows(rb, D),
            _rows(rb, 2 * D),
            _full((8, 2 * D)),
        ],
        out_shape=[
            jax.ShapeDtypeStruct((b, D), odt),
            jax.ShapeDtypeStruct((b, 2 * D), h1dt),
            jax.ShapeDtypeStruct((8, 2 * D), F32),
        ],
    )


@functools.lru_cache(maxsize=None)
def _make_edge_head(b, rb, idt=F32, odt=F32):
    nb = b // rb

    def body(la_ref, lb_ref, rep_ref, wel_ref, bel_ref, w1_ref, sc_ref,
             h1_ref, mom_ref):
        i = pl.program_id(0)
        x = jnp.concatenate([la_ref[...], lb_ref[...]], axis=1).astype(F32)
        y = _dot(x, wel_ref[2 * D:, :])
        c2e = (_dot(x, wel_ref[: 2 * D, :]) + _group_sum_roll(y, 2)
               + bel_ref[0:1, :])
        pre = c2e + sc_ref[0, 0] * rep_ref[...]
        h1 = _dot(pre, w1_ref[...])
        h1_ref[...] = h1.astype(odt)

        @pl.when(i == 0)
        def _():
            mom_ref[...] = jnp.zeros_like(mom_ref)

        mom_ref[0:1, :] += jnp.sum(h1, axis=0, keepdims=True)
        mom_ref[1:2, :] += jnp.sum(h1 * h1, axis=0, keepdims=True)

    return pl.pallas_call(
        body,
        grid=(nb,),
        in_specs=[
            _rows(rb, D),
            _rows(rb, D),
            _rows(rb, D),
            _full((4 * D, D)),
            _full((1, D)),
            _full((D, 2 * D)),
            pl.BlockSpec(memory_space=pltpu.SMEM),
        ],
        out_specs=[
            _rows(rb, 2 * D),
            _full((8, 2 * D)),
        ],
        out_shape=[
            jax.ShapeDtypeStruct((b, 2 * D), odt),
            jax.ShapeDtypeStruct((8, 2 * D), F32),
        ],
    )


@functools.lru_cache(maxsize=None)
def _make_mlp_mid(b, c1, c2, rb, idt=F32, odt=F32):
    nb = b // rb
    inv_n = 1.0 / b

    def body(h1_ref, m1_ref, g_ref, bb_ref, w2_ref, h2_ref, mom_ref):
        i = pl.program_id(0)
        mean = m1_ref[0:1, :] * inv_n
        var = m1_ref[1:2, :] * inv_n - mean * mean
        inv = lax.rsqrt(var + 1e-5)
        a = jnp.maximum(
            (h1_ref[...].astype(F32) - mean) * inv * g_ref[0:1, :]
            + bb_ref[0:1, :], 0.0)
        h2 = _dot(a, w2_ref[...])
        h2_ref[...] = h2.astype(odt)

        @pl.when(i == 0)
        def _():
            mom_ref[...] = jnp.zeros_like(mom_ref)

        mom_ref[0:1, :] += jnp.sum(h2, axis=0, keepdims=True)
        mom_ref[1:2, :] += jnp.sum(h2 * h2, axis=0, keepdims=True)

    return pl.pallas_call(
        body,
        grid=(nb,),
        in_specs=[
            _rows(rb, c1),
            _full((8, c1)),
            _full((1, c1)),
            _full((1, c1)),
            _full((c1, c2)),
        ],
        out_specs=[
            _rows(rb, c2),
            _full((8, c2)),
        ],
        out_shape=[
            jax.ShapeDtypeStruct((b, c2), odt),
            jax.ShapeDtypeStruct((8, c2), F32),
        ],
    )


@functools.lru_cache(maxsize=None)
def _make_mlp_tail(b, c2, rb, idt=F32):
    nb = b // rb
    inv_n = 1.0 / b

    def body(h2_ref, m2_ref, g_ref, bb_ref, o_ref):
        mean = m2_ref[0:1, :] * inv_n
        var = m2_ref[1:2, :] * inv_n - mean * mean
        inv = lax.rsqrt(var + 1e-5)
        o_ref[...] = jnp.maximum(
            (h2_ref[...].astype(F32) - mean) * inv * g_ref[0:1, :]
            + bb_ref[0:1, :], 0.0)

    return pl.pallas_call(
        body,
        grid=(nb,),
        in_specs=[
            _rows(rb, c2),
            _full((8, c2)),
            _full((1, c2)),
            _full((1, c2)),
        ],
        out_specs=_rows(rb, c2),
        out_shape=jax.ShapeDtypeStruct((b, c2), F32),
    )


def kernel(edge_rep, cycle5_rep, cycle6_rep, We1, ge1, be1, We2, ge2, be2,
           Wc1, gc1, bc1, Wc2, gc2, bc2, Wel, bel, Wl5, bl5, Wl6, bl6,
           edge_eps, cycle_eps, edge_nodes, cycle5_nodes, cycle6_nodes):
    be_, b5, b6 = edge_rep.shape[0], cycle5_rep.shape[0], cycle6_rep.shape[0]
    en = edge_nodes.astype(jnp.int32)
    c5n = cycle5_nodes.astype(jnp.int32)
    c6n = cycle6_nodes.astype(jnp.int32)
    zero = jnp.zeros((N, D), F32)
    e_sc = (1.0 + edge_eps).reshape(1, 1).astype(F32)
    c_sc = (1.0 + cycle_eps).reshape(1, 1).astype(F32)
    row1 = lambda v: v.reshape(1, -1).astype(F32)

    bf16 = jnp.bfloat16
    zbf = jnp.zeros((N, D), bf16)

    A = _make_segsum((be_,), D)(zero, edge_rep, en)
    A_rep = _make_segsum((b5, b6), D)(zero, cycle5_rep, c5n, cycle6_rep, c6n)

    x5a, x5b = _make_gather_x(b5, 5)(A, c5n)
    x6a, x6b = _make_gather_x(b6, 6)(A, c6n)

    A5a = _make_segsum((b5,), D)(zero, x5a, c5n)
    A5b = _make_segsum((b5,), D)(zero, x5b, c5n)
    A6a = _make_segsum((b6,), D)(zero, x6a, c6n)
    A6b = _make_segsum((b6,), D)(zero, x6b, c6n)
    l5a = _make_gather(b5, D)(A5a, c5n)
    l5b = _make_gather(b5, D)(A5b, c5n)
    l6a = _make_gather(b6, D)(A6a, c6n)
    l6b = _make_gather(b6, D)(A6b, c6n)

    Wl5f = Wl5.astype(F32)
    Wl6f = Wl6.astype(F32)
    loc_b = _make_gather(be_, D)(A_rep, en)
    bf16 = jnp.bfloat16
    e2c5, h1_5, m5 = _make_cycle_head(b5, 5, 2000, F32, F32, bf16)(
        l5a, l5b, cycle5_rep, Wl5f, row1(bl5), Wc1, c_sc)
    e2c6, h1_6, m6 = _make_cycle_head(b6, 6, 2400, F32, F32, bf16)(
        l6a, l6b, cycle6_rep, Wl6f, row1(bl6), Wc1, c_sc)

    A_e2c = _make_segsum((b5, b6), D)(zero, e2c5, c5n, e2c6, c6n)
    loc_a = _make_gather(be_, D)(A_e2c, en)

    h2_5, m52 = _make_mlp_mid(b5, 2 * D, D, 5000, bf16, bf16)(
        h1_5, m5, row1(gc1), row1(bc1), Wc2)
    c5_out = _make_mlp_tail(b5, D, 5000, bf16)(
        h2_5, m52, row1(gc2), row1(bc2))

    h2_6, m62 = _make_mlp_mid(b6, 2 * D, D, 6000, bf16, bf16)(
        h1_6, m6, row1(gc1), row1(bc1), Wc2)
    c6_out = _make_mlp_tail(b6, D, 6000, bf16)(
        h2_6, m62, row1(gc2), row1(bc2))

    h1_e, me = _make_edge_head(be_, 4000, F32, bf16)(
        loc_a, loc_b, edge_rep, Wel.astype(F32), row1(bel), We1, e_sc)

    h2_e, me2 = _make_mlp_mid(be_, 2 * D, D, 4000, bf16, bf16)(
        h1_e, me, row1(ge1), row1(be1), We2)
    edge_out = _make_mlp_tail(be_, D, 4000, bf16)(
        h2_e, me2, row1(ge2), row1(be2))

    return (edge_out, c5_out, c6_out)
